# Initial kernel scaffold; baseline (speedup 1.0000x reference)
import functools

import jax
import jax.numpy as jnp
from jax import lax
from jax.experimental import pallas as pl
from jax.experimental.pallas import tpu as pltpu
from jax.experimental.pallas import tpu_sc as plsc

_N = 8388608
_NB = 512
_NC = 2
_NS = 16
_NW = _NC * _NS
_L = 16
_PER_TILE = _N // _NW
_CH = 8192
_BANK = _NS * _NB
_ACC_WORDS = 4 * _BANK


def _stats_kernel(yp_hbm, yt_hbm, b_hbm, out_hbm, b_v, yt_v, yp_v, acc, ostage):
    wid = lax.axis_index("s") * _NC + lax.axis_index("c")
    base = wid * _PER_TILE
    lane_base = lax.iota(jnp.int32, _L) * _NB
    ones = jnp.ones((_L,), jnp.float32)
    zeros = jnp.zeros((_L,), jnp.float32)

    def zero_body(i, carry):
        acc[pl.ds(i * _L, _L)] = zeros
        return carry

    lax.fori_loop(0, _ACC_WORDS // _L, zero_body, None)

    def chunk_body(g, carry):
        off = base + g * _CH
        pltpu.sync_copy(b_hbm.at[pl.ds(off, _CH)], b_v)
        pltpu.sync_copy(yt_hbm.at[pl.ds(off, _CH)], yt_v)
        pltpu.sync_copy(yp_hbm.at[pl.ds(off, _CH)], yp_v)

        def vec_body(k, c2):
            s = pl.ds(k * _L, _L)
            idx = lane_base + b_v[s]
            yt = yt_v[s]
            yp = yp_v[s]
            d = yt - yp
            plsc.addupdate_scatter(acc, [idx], ones)
            plsc.addupdate_scatter(acc, [idx + _BANK], yt)
            plsc.addupdate_scatter(acc, [idx + 2 * _BANK], yt * yt)
            plsc.addupdate_scatter(acc, [idx + 3 * _BANK], d * d)
            return c2

        lax.fori_loop(0, _CH // _L, vec_body, None)
        return carry

    lax.fori_loop(0, _PER_TILE // _CH, chunk_body, None)

    for st in range(4):
        def red_body(c, carry, st=st):
            v = acc[pl.ds(st * _BANK + c * _L, _L)]
            for l in range(1, _NS):
                v = v + acc[pl.ds(st * _BANK + l * _NB + c * _L, _L)]
            ostage[pl.ds(st * _NB + c * _L, _L)] = v
            return carry

        lax.fori_loop(0, _NB // _L, red_body, None)

    pltpu.sync_copy(ostage, out_hbm.at[pl.ds(wid * 4 * _NB, 4 * _NB)])


@jax.jit
def _partials(y_pred, y_true, basin):
    mesh = plsc.VectorSubcoreMesh(core_axis_name="c", subcore_axis_name="s")
    return pl.kernel(
        _stats_kernel,
        out_type=jax.ShapeDtypeStruct((_NW * 4 * _NB,), jnp.float32),
        mesh=mesh,
        scratch_types=[
            pltpu.VMEM((_CH,), jnp.int32),
            pltpu.VMEM((_CH,), jnp.float32),
            pltpu.VMEM((_CH,), jnp.float32),
            pltpu.VMEM((_ACC_WORDS,), jnp.float32),
            pltpu.VMEM((4 * _NB,), jnp.float32),
        ],
    )(y_pred, y_true, basin)


def kernel(y_pred, y_true, basin):
    p = _partials(y_pred, y_true, basin).reshape(_NW, 4, _NB).sum(axis=0)
    counts, s1, s2, ssres = p[0], p[1], p[2], p[3]
    present = counts > 0
    ss_tot = s2 - s1 * s1 / counts
    nse = 1.0 - ssres / (ss_tot + 1e-10)
    nse = jnp.where(present, nse, jnp.zeros_like(nse))
    return jnp.sum(nse) / jnp.sum(present)


# speedup vs baseline: 370.2158x; 370.2158x over previous
import functools

import jax
import jax.numpy as jnp
from jax import lax
from jax.experimental import pallas as pl
from jax.experimental.pallas import tpu as pltpu
from jax.experimental.pallas import tpu_sc as plsc

_N = 8388608
_NB = 512
_NC = 2
_NS = 16
_NW = _NC * _NS
_L = 16
_PER_TILE = _N // _NW
_CH = 8192
_BANK = _NS * _NB
_ACC_WORDS = 4 * _BANK


def _stats_kernel(yp_hbm, yt_hbm, b_hbm, out_hbm, b_v, yt_v, yp_v, acc, ostage):
    wid = lax.axis_index("s") * _NC + lax.axis_index("c")
    base = wid * _PER_TILE
    lane_base = lax.iota(jnp.int32, _L) * _NB
    ones = jnp.ones((_L,), jnp.float32)
    zeros = jnp.zeros((_L,), jnp.float32)

    def zero_body(i, carry):
        acc[pl.ds(i * _L, _L)] = zeros
        return carry

    lax.fori_loop(0, _ACC_WORDS // _L, zero_body, None)

    def chunk_body(g, carry):
        off = base + g * _CH
        pltpu.sync_copy(b_hbm.at[pl.ds(off, _CH)], b_v)
        pltpu.sync_copy(yt_hbm.at[pl.ds(off, _CH)], yt_v)
        pltpu.sync_copy(yp_hbm.at[pl.ds(off, _CH)], yp_v)

        def vec_body(k, c2):
            s = pl.ds(k * _L, _L)
            idx = lane_base + b_v[s]
            yt = yt_v[s]
            yp = yp_v[s]
            d = yt - yp
            plsc.addupdate_scatter(acc, [idx], ones)
            plsc.addupdate_scatter(acc, [idx + _BANK], yt)
            plsc.addupdate_scatter(acc, [idx + 2 * _BANK], yt * yt)
            plsc.addupdate_scatter(acc, [idx + 3 * _BANK], d * d)
            return c2

        lax.fori_loop(0, _CH // _L, vec_body, None)
        return carry

    lax.fori_loop(0, _PER_TILE // _CH, chunk_body, None)

    for st in range(4):
        def red_body(c, carry, st=st):
            v = acc[pl.ds(st * _BANK + c * _L, _L)]
            for l in range(1, _NS):
                v = v + acc[pl.ds(st * _BANK + l * _NB + c * _L, _L)]
            ostage[pl.ds(st * _NB + c * _L, _L)] = v
            return carry

        lax.fori_loop(0, _NB // _L, red_body, None)

    pltpu.sync_copy(ostage, out_hbm.at[pl.ds(wid * 4 * _NB, 4 * _NB)])


@jax.jit
def _partials(y_pred, y_true, basin):
    mesh = plsc.VectorSubcoreMesh(core_axis_name="c", subcore_axis_name="s")
    return pl.kernel(
        _stats_kernel,
        out_type=jax.ShapeDtypeStruct((_NW * 4 * _NB,), jnp.float32),
        mesh=mesh,
        compiler_params=pltpu.CompilerParams(needs_layout_passes=False),
        scratch_types=[
            pltpu.VMEM((_CH,), jnp.int32),
            pltpu.VMEM((_CH,), jnp.float32),
            pltpu.VMEM((_CH,), jnp.float32),
            pltpu.VMEM((_ACC_WORDS,), jnp.float32),
            pltpu.VMEM((4 * _NB,), jnp.float32),
        ],
    )(y_pred, y_true, basin)


def kernel(y_pred, y_true, basin):
    p = _partials(y_pred, y_true, basin).reshape(_NW, 4, _NB).sum(axis=0)
    counts, s1, s2, ssres = p[0], p[1], p[2], p[3]
    present = counts > 0
    ss_tot = s2 - s1 * s1 / counts
    nse = 1.0 - ssres / (ss_tot + 1e-10)
    nse = jnp.where(present, nse, jnp.zeros_like(nse))
    return jnp.sum(nse) / jnp.sum(present)
